# baseline (speedup 1.0000x reference)
import functools

import jax
import jax.numpy as jnp
from jax import lax
from jax.experimental import pallas as pl
from jax.experimental.pallas import tpu as pltpu
from jax.experimental.pallas import tpu_sc as plsc

NC = 2
NS = 16
NW = NC * NS
LANES = 16

D = 768
BATCH = 4
SEQ = 8192
SPW = SEQ // NW
CP = 8
NCH = SPW // CP
VPR = D // LANES
NSLOT = 4


def _emb_kernel(x_hbm, tab_hbm, pos_hbm, out_hbm,
                idx_v, pos_v, tok_v, gsem, psem, wsem):
    wid = lax.axis_index("s") * NC + lax.axis_index("c")
    s0 = wid * SPW

    for b in range(BATCH):
        pltpu.sync_copy(x_hbm.at[pl.ds(b * SEQ + s0, SPW)], idx_v.at[b])

    def fire(c, s):
        pltpu.async_copy(
            pos_hbm.at[pl.ds(s0 + c * CP, CP), :], pos_v.at[s], psem.at[s])
        for b in range(BATCH):
            pltpu.async_copy(
                tab_hbm.at[idx_v.at[b, pl.ds(c * CP, CP)]],
                tok_v.at[s, b], gsem.at[s])

    def wait_fire(c, s):
        pltpu.make_async_copy(
            pos_hbm.at[pl.ds(s0 + c * CP, CP), :], pos_v.at[s], psem.at[s]
        ).wait()
        for b in range(BATCH):
            pltpu.make_async_copy(
                tab_hbm.at[idx_v.at[b, pl.ds(c * CP, CP)]],
                tok_v.at[s, b], gsem.at[s]).wait()

    def put(c, s):
        for b in range(BATCH):
            pltpu.async_copy(
                tok_v.at[s, b],
                out_hbm.at[pl.ds(b * SEQ + s0 + c * CP, CP), :], wsem.at[s])

    def wait_put(c, s):
        for b in range(BATCH):
            pltpu.make_async_copy(
                tok_v.at[s, b],
                out_hbm.at[pl.ds(b * SEQ + s0 + c * CP, CP), :], wsem.at[s]
            ).wait()

    fire(0, 0)
    fire(1, 1)

    def step(c, carry):
        s = c & (NSLOT - 1)
        s2 = (c + 2) & (NSLOT - 1)

        @pl.when(c >= 2)
        def _():
            wait_put(c - 2, s2)

        @pl.when(c < NCH - 2)
        def _():
            fire(c + 2, s2)

        wait_fire(c, s)

        def add_row(r, carry2):
            for v in range(VPR):
                sl = pl.ds(v * LANES, LANES)
                pv = pos_v[s, r, sl]
                for b in range(BATCH):
                    tok_v[s, b, r, sl] = tok_v[s, b, r, sl] + pv
            return carry2

        lax.fori_loop(0, CP, add_row, 0)
        put(c, s)
        return carry

    lax.fori_loop(0, NCH, step, 0)

    wait_put(NCH - 2, (NCH - 2) & (NSLOT - 1))
    wait_put(NCH - 1, (NCH - 1) & (NSLOT - 1))


@jax.jit
def _emb(x_flat, tab, pos):
    mesh = plsc.VectorSubcoreMesh(
        core_axis_name="c", subcore_axis_name="s", num_cores=NC, num_subcores=NS
    )
    run = functools.partial(
        pl.kernel,
        out_type=jax.ShapeDtypeStruct((BATCH * SEQ, D), jnp.float32),
        mesh=mesh,
        scratch_types=[
            pltpu.VMEM((BATCH, SPW), jnp.int32),
            pltpu.VMEM((NSLOT, CP, D), jnp.float32),
            pltpu.VMEM((NSLOT, BATCH, CP, D), jnp.float32),
            pltpu.SemaphoreType.DMA((NSLOT,)),
            pltpu.SemaphoreType.DMA((NSLOT,)),
            pltpu.SemaphoreType.DMA((NSLOT,)),
        ],
    )(_emb_kernel)
    return run(x_flat, tab, pos)


def kernel(x, tok_table, pos_emb):
    x_flat = x.reshape(-1).astype(jnp.int32)
    pos = pos_emb[: x.shape[1], :]
    out = _emb(x_flat, tok_table, pos)
    return out.reshape(x.shape[0], x.shape[1], D)


# speedup vs baseline: 2.5088x; 2.5088x over previous
import functools

import jax
import jax.numpy as jnp
from jax import lax
from jax.experimental import pallas as pl
from jax.experimental.pallas import tpu as pltpu
from jax.experimental.pallas import tpu_sc as plsc

NC = 2
NS = 16
NW = NC * NS
LANES = 16

D = 768
BATCH = 4
SEQ = 8192
SPW = SEQ // NW
CP = 8
NCH = SPW // CP
NCB = NCH // 4
VPR = D // LANES
NSLOT = 4


def _emb_kernel(x_hbm, tab_hbm, pos_hbm, out_hbm,
                idx_v, pos_v, tok_v, gsem, psem, wsem):
    wid = lax.axis_index("s") * NC + lax.axis_index("c")
    s0 = wid * SPW

    pltpu.sync_copy(x_hbm.at[wid], idx_v)

    def fire(c, s):
        pltpu.async_copy(
            pos_hbm.at[pl.ds(s0 + c * CP, CP), :], pos_v.at[s], psem.at[s])
        pltpu.async_copy(tab_hbm.at[idx_v.at[c]], tok_v.at[s], gsem.at[s])

    def wait_fire(c, s):
        pltpu.make_async_copy(
            pos_hbm.at[pl.ds(s0 + c * CP, CP), :], pos_v.at[s], psem.at[s]
        ).wait()
        pltpu.make_async_copy(
            tab_hbm.at[idx_v.at[c]], tok_v.at[s], gsem.at[s]).wait()

    def put(c, s):
        for b in range(BATCH):
            pltpu.async_copy(
                tok_v.at[s, pl.ds(b * CP, CP)],
                out_hbm.at[pl.ds(b * SEQ + s0 + c * CP, CP), :], wsem.at[s])

    def wait_put(c, s):
        for b in range(BATCH):
            pltpu.make_async_copy(
                tok_v.at[s, pl.ds(b * CP, CP)],
                out_hbm.at[pl.ds(b * SEQ + s0 + c * CP, CP), :], wsem.at[s]
            ).wait()

    def process(c, s):
        def add_row(r, carry2):
            for v in range(VPR):
                sl = pl.ds(v * LANES, LANES)
                pv = pos_v[s, r, sl]
                for b in range(BATCH):
                    tok_v[s, b * CP + r, sl] = tok_v[s, b * CP + r, sl] + pv
            return carry2

        lax.fori_loop(0, CP, add_row, 0)

    fire(0, 0)
    fire(1, 1)

    def outer(cb, carry):
        for k in range(4):
            c = cb * 4 + k
            s = k
            s2 = (k + 2) % 4
            if k < 2:
                @pl.when(cb >= 1)
                def _():
                    wait_put(c - 2, s2)
                fire(c + 2, s2)
            else:
                wait_put(c - 2, s2)

                @pl.when(cb <= NCB - 2)
                def _():
                    fire(c + 2, s2)
            wait_fire(c, s)
            process(c, s)
            put(c, s)
        return carry

    lax.fori_loop(0, NCB, outer, 0)

    wait_put(NCH - 2, 2)
    wait_put(NCH - 1, 3)


@jax.jit
def _emb(x4, tab, pos):
    mesh = plsc.VectorSubcoreMesh(
        core_axis_name="c", subcore_axis_name="s", num_cores=NC, num_subcores=NS
    )
    run = functools.partial(
        pl.kernel,
        out_type=jax.ShapeDtypeStruct((BATCH * SEQ, D), jnp.float32),
        mesh=mesh,
        scratch_types=[
            pltpu.VMEM((NCH, BATCH * CP), jnp.int32),
            pltpu.VMEM((NSLOT, CP, D), jnp.float32),
            pltpu.VMEM((NSLOT, BATCH * CP, D), jnp.float32),
            pltpu.SemaphoreType.DMA((NSLOT,)),
            pltpu.SemaphoreType.DMA((NSLOT,)),
            pltpu.SemaphoreType.DMA((NSLOT,)),
        ],
    )(_emb_kernel)
    return run(x4, tab, pos)


def kernel(x, tok_table, pos_emb):
    x4 = (
        x.astype(jnp.int32)
        .reshape(BATCH, NW, NCH, CP)
        .transpose(1, 2, 0, 3)
        .reshape(NW, NCH, BATCH * CP)
    )
    pos = pos_emb[: x.shape[1], :]
    out = _emb(x4, tok_table, pos)
    return out.reshape(x.shape[0], x.shape[1], D)
